# baseline (speedup 1.0000x reference)
import functools

import jax
import jax.numpy as jnp
from jax.experimental import pallas as pl
from jax.experimental.pallas import tpu as pltpu

H = 32
ED = 16


def _silu(v):
    return v * jax.nn.sigmoid(v)


def _edge_mlp_body(hi_ref, hj_ref, ea_ref, dist_ref,
                   w1h_i_ref, w1h_j_ref, w1e_ref, w1d_ref, b1_ref,
                   w2_ref, b2_ref, wx1_ref, bx1_ref, wx2_ref, bx2_ref,
                   m_ref, g_ref):
    hi = hi_ref[...]
    hj = hj_ref[...]
    ea = ea_ref[...]
    dist = dist_ref[...]
    pre1 = (jnp.dot(hi, w1h_i_ref[...], preferred_element_type=jnp.float32)
            + jnp.dot(hj, w1h_j_ref[...], preferred_element_type=jnp.float32)
            + jnp.dot(ea, w1e_ref[...], preferred_element_type=jnp.float32)
            + dist * w1d_ref[...]
            + b1_ref[...])
    m1 = _silu(pre1)
    m = jnp.dot(m1, w2_ref[...], preferred_element_type=jnp.float32) + b2_ref[...]
    g1 = _silu(jnp.dot(m, wx1_ref[...], preferred_element_type=jnp.float32) + bx1_ref[...])
    gate = jnp.dot(g1, wx2_ref[...], preferred_element_type=jnp.float32) + bx2_ref[...]
    m_ref[...] = m
    g_ref[...] = gate


def _edge_mlp(p, hi, hj, ea, dist, block=8000):
    E = hi.shape[0]
    (w1, b1), (w2, b2) = p['phi_e']
    (wx1, bx1), (wx2, bx2) = p['phi_x']
    w1h_i = w1[:H]
    w1h_j = w1[H:2 * H]
    w1e = w1[2 * H:2 * H + ED]
    w1d = w1[2 * H + ED:]
    grid = (E // block,)
    ew = lambda *_: (0, 0)
    m, g = pl.pallas_call(
        _edge_mlp_body,
        grid=grid,
        in_specs=[
            pl.BlockSpec((block, H), lambda i: (i, 0)),
            pl.BlockSpec((block, H), lambda i: (i, 0)),
            pl.BlockSpec((block, ED), lambda i: (i, 0)),
            pl.BlockSpec((block, 1), lambda i: (i, 0)),
            pl.BlockSpec((H, H), ew),
            pl.BlockSpec((H, H), ew),
            pl.BlockSpec((ED, H), ew),
            pl.BlockSpec((1, H), ew),
            pl.BlockSpec((1, H), ew),
            pl.BlockSpec((H, H), ew),
            pl.BlockSpec((1, H), ew),
            pl.BlockSpec((H, H), ew),
            pl.BlockSpec((1, H), ew),
            pl.BlockSpec((H, 1), ew),
            pl.BlockSpec((1, 1), ew),
        ],
        out_specs=[
            pl.BlockSpec((block, H), lambda i: (i, 0)),
            pl.BlockSpec((block, 1), lambda i: (i, 0)),
        ],
        out_shape=[
            jax.ShapeDtypeStruct((E, H), jnp.float32),
            jax.ShapeDtypeStruct((E, 1), jnp.float32),
        ],
    )(hi, hj, ea, dist,
      w1h_i, w1h_j, w1e, w1d, b1[None, :],
      w2, b2[None, :], wx1, bx1[None, :], wx2, bx2[None, :])
    return m, g


def _node_update_body(h_ref, agg_ref, w1h_ref, w1a_ref, b1_ref, w2_ref, b2_ref,
                      g_ref, bln_ref, out_ref):
    h = h_ref[...]
    agg = agg_ref[...]
    d1 = _silu(jnp.dot(h, w1h_ref[...], preferred_element_type=jnp.float32)
               + jnp.dot(agg, w1a_ref[...], preferred_element_type=jnp.float32)
               + b1_ref[...])
    dh = jnp.dot(d1, w2_ref[...], preferred_element_type=jnp.float32) + b2_ref[...]
    hn = h + dh
    mu = jnp.mean(hn, axis=-1, keepdims=True)
    var = jnp.mean((hn - mu) * (hn - mu), axis=-1, keepdims=True)
    out_ref[...] = (hn - mu) * jax.lax.rsqrt(var + 1e-05) * g_ref[...] + bln_ref[...]


def _node_update(p, h, agg, block=5000):
    N = h.shape[0]
    (w1, b1), (w2, b2) = p['phi_h']
    w1h = w1[:H]
    w1a = w1[H:]
    pad = (-N) % block
    if pad:
        h = jnp.pad(h, ((0, pad), (0, 0)))
        agg = jnp.pad(agg, ((0, pad), (0, 0)))
    Np = h.shape[0]
    ew = lambda *_: (0, 0)
    out = pl.pallas_call(
        _node_update_body,
        grid=(Np // block,),
        in_specs=[
            pl.BlockSpec((block, H), lambda i: (i, 0)),
            pl.BlockSpec((block, H), lambda i: (i, 0)),
            pl.BlockSpec((H, H), ew),
            pl.BlockSpec((H, H), ew),
            pl.BlockSpec((1, H), ew),
            pl.BlockSpec((H, H), ew),
            pl.BlockSpec((1, H), ew),
            pl.BlockSpec((1, H), ew),
            pl.BlockSpec((1, H), ew),
        ],
        out_specs=pl.BlockSpec((block, H), lambda i: (i, 0)),
        out_shape=jax.ShapeDtypeStruct((Np, H), jnp.float32),
    )(h, agg, w1h, w1a, b1[None, :], w2, b2[None, :],
      p['ln_g'][None, :], p['ln_b'][None, :])
    return out[:N]


def _egnn_layer(p, h, x, src, dst, edge_attr, N):
    hi = h[src]
    hj = h[dst]
    xi = x[src]
    xj = x[dst]
    diff = xj - xi
    sq = jnp.sum(diff * diff, axis=-1, keepdims=True)
    dist = jnp.sqrt(sq + 1e-09)
    m, gate = _edge_mlp(p, hi, hj, edge_attr, dist)
    agg = jax.ops.segment_sum(m, dst, num_segments=N)
    h_out = _node_update(p, h, agg)
    dnorm = jnp.sqrt(sq) + 1e-09
    coord_msg = diff / dnorm * gate
    dx = jax.ops.segment_sum(coord_msg, dst, num_segments=N)
    return h_out, x + dx


def _pool(h, batch, nb):
    s = jax.ops.segment_sum(h, batch, num_segments=nb)
    c = jax.ops.segment_sum(jnp.ones((h.shape[0], 1), dtype=h.dtype), batch, num_segments=nb)
    return s / jnp.clip(c, 1.0, None)


def _mlp(params, v):
    n = len(params)
    for i, (W, b) in enumerate(params):
        v = v @ W + b
        if i < n - 1:
            v = _silu(v)
    return v


def kernel(prot_pos, prot_feat, prot_edge_index, prot_edge_attr, lig_pos, lig_feat,
           lig_edge_index, lig_edge_attr, prot_batch, lig_batch, priors,
           prot_proj_W, prot_proj_b, lig_proj_W, lig_proj_b,
           prot_layers, lig_layers, readout):
    NP = prot_pos.shape[0]
    NL = lig_pos.shape[0]
    h_p = prot_feat @ prot_proj_W + prot_proj_b
    x_p = prot_pos
    psrc = prot_edge_index[0]
    pdst = prot_edge_index[1]
    for p in prot_layers:
        h_p, x_p = _egnn_layer(p, h_p, x_p, psrc, pdst, prot_edge_attr, NP)
    h_l = lig_feat @ lig_proj_W + lig_proj_b
    x_l = lig_pos
    lsrc = lig_edge_index[0]
    ldst = lig_edge_index[1]
    for p in lig_layers:
        h_l, x_l = _egnn_layer(p, h_l, x_l, lsrc, ldst, lig_edge_attr, NL)
    B = priors.shape[0]
    prot_pool = _pool(h_p, prot_batch, B)
    lig_pool = _pool(h_l, lig_batch, B)
    complex_feat = jnp.concatenate([prot_pool, lig_pool, priors], axis=-1)
    pred = _mlp(readout, complex_feat)
    return pred[:, 0]


# speedup vs baseline: 1.0840x; 1.0840x over previous
import functools

import jax
import jax.numpy as jnp
from jax import lax
from jax.experimental import pallas as pl
from jax.experimental.pallas import tpu as pltpu
from jax.experimental.pallas import tpu_sc as plsc

H = 32
ED = 16
VW = 40
CH = 1024
NWORK = 32


def _silu(v):
    return v * jax.nn.sigmoid(v)


def _edge_body(hi_ref, hj_ref, ea_ref, dist_ref, dn3_ref,
               w1h_i_ref, w1h_j_ref, w1e_ref, w1d_ref, b1_ref,
               w2_ref, b2_ref, wx1_ref, bx1_ref, wx2_ref, bx2_ref,
               out_ref):
    hi = hi_ref[...]
    hj = hj_ref[...]
    ea = ea_ref[...]
    dist = dist_ref[...]
    pre1 = (jnp.dot(hi, w1h_i_ref[...], preferred_element_type=jnp.float32)
            + jnp.dot(hj, w1h_j_ref[...], preferred_element_type=jnp.float32)
            + jnp.dot(ea, w1e_ref[...], preferred_element_type=jnp.float32)
            + dist * w1d_ref[...]
            + b1_ref[...])
    m1 = _silu(pre1)
    m = jnp.dot(m1, w2_ref[...], preferred_element_type=jnp.float32) + b2_ref[...]
    g1 = _silu(jnp.dot(m, wx1_ref[...], preferred_element_type=jnp.float32) + bx1_ref[...])
    gate = jnp.dot(g1, wx2_ref[...], preferred_element_type=jnp.float32) + bx2_ref[...]
    cmsg = dn3_ref[...] * gate
    blk = hi.shape[0]
    out_ref[...] = jnp.concatenate(
        [m, cmsg, jnp.zeros((blk, VW - H - 3), jnp.float32)], axis=1)


def _edge_mlp(p, hi, hj, ea, dist, dn3, epad, block=8000):
    E = hi.shape[0]
    (w1, b1), (w2, b2) = p['phi_e']
    (wx1, bx1), (wx2, bx2) = p['phi_x']
    ew = lambda *_: (0, 0)
    out = pl.pallas_call(
        _edge_body,
        grid=(E // block,),
        in_specs=[
            pl.BlockSpec((block, H), lambda i: (i, 0)),
            pl.BlockSpec((block, H), lambda i: (i, 0)),
            pl.BlockSpec((block, ED), lambda i: (i, 0)),
            pl.BlockSpec((block, 1), lambda i: (i, 0)),
            pl.BlockSpec((block, 3), lambda i: (i, 0)),
            pl.BlockSpec((H, H), ew),
            pl.BlockSpec((H, H), ew),
            pl.BlockSpec((ED, H), ew),
            pl.BlockSpec((1, H), ew),
            pl.BlockSpec((1, H), ew),
            pl.BlockSpec((H, H), ew),
            pl.BlockSpec((1, H), ew),
            pl.BlockSpec((H, H), ew),
            pl.BlockSpec((1, H), ew),
            pl.BlockSpec((H, 1), ew),
            pl.BlockSpec((1, 1), ew),
        ],
        out_specs=pl.BlockSpec((block, VW), lambda i: (i, 0)),
        out_shape=jax.ShapeDtypeStruct((epad, VW), jnp.float32),
    )(hi, hj, ea, dist, dn3,
      w1[:H], w1[H:2 * H], w1[2 * H:2 * H + ED], w1[2 * H + ED:], b1[None, :],
      w2, b2[None, :], wx1, bx1[None, :], wx2, bx2[None, :])
    return out


def _sc_scatter(vals, idx2d, zrows, nt):
    rows = idx2d.shape[0]
    per_w = rows // NWORK
    n_chunks = per_w // 2
    zr = nt // 16
    mesh = plsc.VectorSubcoreMesh(core_axis_name="c", subcore_axis_name="s")

    @functools.partial(
        pl.kernel,
        mesh=mesh,
        compiler_params=pltpu.CompilerParams(use_tc_tiling_on_sc=False),
        out_type=jax.ShapeDtypeStruct((2, nt, VW), jnp.float32),
        scratch_types=[
            pltpu.VMEM((2, 128), jnp.int32),
            pltpu.VMEM((128, VW), jnp.float32),
            pltpu.VMEM_SHARED((nt, VW), jnp.float32),
        ],
    )
    def k(vals_hbm, idx_hbm, z_hbm, out_hbm, idx_v, vals_v, shared):
        c = lax.axis_index("c")
        s = lax.axis_index("s")
        wid = s * 2 + c
        pltpu.sync_copy(z_hbm, shared.at[pl.ds(s * zr, zr)])
        plsc.subcore_barrier()

        def body(t, carry):
            row0 = wid * per_w + t * 2
            pltpu.sync_copy(idx_hbm.at[pl.ds(row0, 2)], idx_v)
            for j in range(2):
                pltpu.sync_copy(vals_hbm.at[pl.ds((row0 + j) * 128, 128)], vals_v)
                pltpu.sync_copy(vals_v, shared.at[idx_v.at[j]], add=True)
            return carry

        lax.fori_loop(0, n_chunks, body, 0)
        plsc.subcore_barrier()
        pltpu.sync_copy(shared.at[pl.ds(s * zr, zr)],
                        out_hbm.at[c, pl.ds(s * zr, zr)])

    return k(vals, idx2d, zrows)


def _node_body(h_ref, p0_ref, p1_ref, w1h_ref, w1a_ref, b1_ref, w2_ref, b2_ref,
               g_ref, bln_ref, out_ref):
    h = h_ref[...]
    agg = p0_ref[...][:, :H] + p1_ref[...][:, :H]
    d1 = _silu(jnp.dot(h, w1h_ref[...], preferred_element_type=jnp.float32)
               + jnp.dot(agg, w1a_ref[...], preferred_element_type=jnp.float32)
               + b1_ref[...])
    dh = jnp.dot(d1, w2_ref[...], preferred_element_type=jnp.float32) + b2_ref[...]
    hn = h + dh
    mu = jnp.mean(hn, axis=-1, keepdims=True)
    var = jnp.mean((hn - mu) * (hn - mu), axis=-1, keepdims=True)
    out_ref[...] = (hn - mu) * lax.rsqrt(var + 1e-05) * g_ref[...] + bln_ref[...]


def _node_update(p, h, p0, p1, block=5000):
    N = h.shape[0]
    (w1, b1), (w2, b2) = p['phi_h']
    ew = lambda *_: (0, 0)
    out = pl.pallas_call(
        _node_body,
        grid=(N // block,),
        in_specs=[
            pl.BlockSpec((block, H), lambda i: (i, 0)),
            pl.BlockSpec((block, VW), lambda i: (i, 0)),
            pl.BlockSpec((block, VW), lambda i: (i, 0)),
            pl.BlockSpec((H, H), ew),
            pl.BlockSpec((H, H), ew),
            pl.BlockSpec((1, H), ew),
            pl.BlockSpec((H, H), ew),
            pl.BlockSpec((1, H), ew),
            pl.BlockSpec((1, H), ew),
            pl.BlockSpec((1, H), ew),
        ],
        out_specs=pl.BlockSpec((block, H), lambda i: (i, 0)),
        out_shape=jax.ShapeDtypeStruct((N, H), jnp.float32),
    )(h, p0, p1, w1[:H], w1[H:], b1[None, :], w2, b2[None, :],
      p['ln_g'][None, :], p['ln_b'][None, :])
    return out


def _egnn_layer(p, h, x, src, dst, idx2d, zrows, nt, epad, edge_attr, N):
    hi = h[src]
    hj = h[dst]
    xi = x[src]
    xj = x[dst]
    diff = xj - xi
    sq = jnp.sum(diff * diff, axis=-1, keepdims=True)
    dist = jnp.sqrt(sq + 1e-09)
    dn3 = diff / (jnp.sqrt(sq) + 1e-09)
    vals = _edge_mlp(p, hi, hj, edge_attr, dist, dn3, epad)
    part = _sc_scatter(vals, idx2d, zrows, nt)
    h_out = _node_update(p, h, part[0, :N], part[1, :N])
    dx = part[0, :N, H:H + 3] + part[1, :N, H:H + 3]
    return h_out, x + dx


def _pool(h, batch, nb):
    s = jax.ops.segment_sum(h, batch, num_segments=nb)
    c = jax.ops.segment_sum(jnp.ones((h.shape[0], 1), dtype=h.dtype), batch, num_segments=nb)
    return s / jnp.clip(c, 1.0, None)


def _mlp(params, v):
    n = len(params)
    for i, (W, b) in enumerate(params):
        v = v @ W + b
        if i < n - 1:
            v = _silu(v)
    return v


def _graph_stack(layers, h, x, src, dst, edge_attr, N):
    E = src.shape[0]
    epad = ((E + NWORK * 8 * 128 - 1) // (NWORK * 8 * 128)) * (NWORK * 8 * 128)
    nt = ((N + 1 + 127) // 128) * 128
    dst_pad = jnp.pad(dst.astype(jnp.int32), (0, epad - E), constant_values=N)
    idx2d = dst_pad.reshape(epad // 128, 128)
    zrows = jnp.zeros((nt // 16, VW), jnp.float32)
    for p in layers:
        h, x = _egnn_layer(p, h, x, src, dst, idx2d, zrows, nt, epad, edge_attr, N)
    return h, x


def kernel(prot_pos, prot_feat, prot_edge_index, prot_edge_attr, lig_pos, lig_feat,
           lig_edge_index, lig_edge_attr, prot_batch, lig_batch, priors,
           prot_proj_W, prot_proj_b, lig_proj_W, lig_proj_b,
           prot_layers, lig_layers, readout):
    NP = prot_pos.shape[0]
    NL = lig_pos.shape[0]
    h_p = prot_feat @ prot_proj_W + prot_proj_b
    h_p, _ = _graph_stack(prot_layers, h_p, prot_pos,
                          prot_edge_index[0], prot_edge_index[1],
                          prot_edge_attr, NP)
    h_l = lig_feat @ lig_proj_W + lig_proj_b
    h_l, _ = _graph_stack(lig_layers, h_l, lig_pos,
                          lig_edge_index[0], lig_edge_index[1],
                          lig_edge_attr, NL)
    B = priors.shape[0]
    prot_pool = _pool(h_p, prot_batch, B)
    lig_pool = _pool(h_l, lig_batch, B)
    complex_feat = jnp.concatenate([prot_pool, lig_pool, priors], axis=-1)
    pred = _mlp(readout, complex_feat)
    return pred[:, 0]


# speedup vs baseline: 3.5997x; 3.3208x over previous
import functools

import jax
import jax.numpy as jnp
from jax import lax
from jax.experimental import pallas as pl
from jax.experimental.pallas import tpu as pltpu
from jax.experimental.pallas import tpu_sc as plsc

H = 32
ED = 16
VW = 40
CH = 1024
NWORK = 32


def _silu(v):
    return v * jax.nn.sigmoid(v)


CW = 48


def _sc_gather(comb, idx2d, epad):
    rows = idx2d.shape[0]
    per_w = rows // NWORK
    n_chunks = per_w // 4
    mesh = plsc.VectorSubcoreMesh(core_axis_name="c", subcore_axis_name="s")

    @functools.partial(
        pl.kernel,
        mesh=mesh,
        compiler_params=pltpu.CompilerParams(use_tc_tiling_on_sc=False),
        out_type=jax.ShapeDtypeStruct((2 * epad, CW), jnp.float32),
        scratch_types=[
            pltpu.VMEM((4, 128), jnp.int32),
            pltpu.VMEM((512, CW), jnp.float32),
            pltpu.SemaphoreType.DMA,
        ],
    )
    def k(comb_hbm, idx_hbm, out_hbm, idx_v, rows_v, sem):
        c = lax.axis_index("c")
        s = lax.axis_index("s")
        wid = s * 2 + c

        def body(t, carry):
            row0 = wid * per_w + t * 4
            pltpu.sync_copy(idx_hbm.at[pl.ds(row0, 4)], idx_v)
            for j in range(4):
                pltpu.sync_copy(comb_hbm.at[idx_v.at[j]],
                                rows_v.at[pl.ds(j * 128, 128)])
            pltpu.sync_copy(rows_v, out_hbm.at[pl.ds(row0 * 128, 512)])
            return carry

        lax.fori_loop(0, n_chunks, body, 0)

    out = k(comb, idx2d)
    return out.reshape(2, epad, CW)


def _edge_body(g0_ref, g1_ref, ea_ref,
               w1h_i_ref, w1h_j_ref, w1e_ref, w1d_ref, b1_ref,
               w2_ref, b2_ref, wx1_ref, bx1_ref, wx2_ref, bx2_ref,
               out_ref):
    g0 = g0_ref[0]
    g1 = g1_ref[0]
    hi = g0[:, :H]
    hj = g1[:, :H]
    xi = g0[:, H:H + 3]
    xj = g1[:, H:H + 3]
    diff = xj - xi
    sq = jnp.sum(diff * diff, axis=-1, keepdims=True)
    dist = jnp.sqrt(sq + 1e-09)
    dn3 = diff / (jnp.sqrt(sq) + 1e-09)
    ea = ea_ref[...]
    pre1 = (jnp.dot(hi, w1h_i_ref[...], preferred_element_type=jnp.float32)
            + jnp.dot(hj, w1h_j_ref[...], preferred_element_type=jnp.float32)
            + jnp.dot(ea, w1e_ref[...], preferred_element_type=jnp.float32)
            + dist * w1d_ref[...]
            + b1_ref[...])
    m1 = _silu(pre1)
    m = jnp.dot(m1, w2_ref[...], preferred_element_type=jnp.float32) + b2_ref[...]
    gg = _silu(jnp.dot(m, wx1_ref[...], preferred_element_type=jnp.float32) + bx1_ref[...])
    gate = jnp.dot(gg, wx2_ref[...], preferred_element_type=jnp.float32) + bx2_ref[...]
    cmsg = dn3 * gate
    blk = hi.shape[0]
    out_ref[...] = jnp.concatenate(
        [m, cmsg, jnp.zeros((blk, VW - H - 3), jnp.float32)], axis=1)


def _edge_mlp(p, gathered, ea, epad, block=8000):
    E = ea.shape[0]
    (w1, b1), (w2, b2) = p['phi_e']
    (wx1, bx1), (wx2, bx2) = p['phi_x']
    ew = lambda *_: (0, 0)
    out = pl.pallas_call(
        _edge_body,
        grid=(E // block,),
        in_specs=[
            pl.BlockSpec((1, block, CW), lambda i: (0, i, 0)),
            pl.BlockSpec((1, block, CW), lambda i: (1, i, 0)),
            pl.BlockSpec((block, ED), lambda i: (i, 0)),
            pl.BlockSpec((H, H), ew),
            pl.BlockSpec((H, H), ew),
            pl.BlockSpec((ED, H), ew),
            pl.BlockSpec((1, H), ew),
            pl.BlockSpec((1, H), ew),
            pl.BlockSpec((H, H), ew),
            pl.BlockSpec((1, H), ew),
            pl.BlockSpec((H, H), ew),
            pl.BlockSpec((1, H), ew),
            pl.BlockSpec((H, 1), ew),
            pl.BlockSpec((1, 1), ew),
        ],
        out_specs=pl.BlockSpec((block, VW), lambda i: (i, 0)),
        out_shape=jax.ShapeDtypeStruct((epad, VW), jnp.float32),
    )(gathered, gathered, ea,
      w1[:H], w1[H:2 * H], w1[2 * H:2 * H + ED], w1[2 * H + ED:], b1[None, :],
      w2, b2[None, :], wx1, bx1[None, :], wx2, bx2[None, :])
    return out


def _sc_scatter(vals, idx2d, zrows, nt):
    rows = idx2d.shape[0]
    per_w = rows // NWORK
    n_chunks = per_w // 2
    zr = nt // 16
    mesh = plsc.VectorSubcoreMesh(core_axis_name="c", subcore_axis_name="s")

    @functools.partial(
        pl.kernel,
        mesh=mesh,
        compiler_params=pltpu.CompilerParams(use_tc_tiling_on_sc=False),
        out_type=jax.ShapeDtypeStruct((2, nt, VW), jnp.float32),
        scratch_types=[
            pltpu.VMEM((2, 128), jnp.int32),
            pltpu.VMEM((128, VW), jnp.float32),
            pltpu.VMEM_SHARED((nt, VW), jnp.float32),
        ],
    )
    def k(vals_hbm, idx_hbm, z_hbm, out_hbm, idx_v, vals_v, shared):
        c = lax.axis_index("c")
        s = lax.axis_index("s")
        wid = s * 2 + c
        pltpu.sync_copy(z_hbm, shared.at[pl.ds(s * zr, zr)])
        plsc.subcore_barrier()

        def body(t, carry):
            row0 = wid * per_w + t * 2
            pltpu.sync_copy(idx_hbm.at[pl.ds(row0, 2)], idx_v)
            for j in range(2):
                pltpu.sync_copy(vals_hbm.at[pl.ds((row0 + j) * 128, 128)], vals_v)
                pltpu.sync_copy(vals_v, shared.at[idx_v.at[j]], add=True)
            return carry

        lax.fori_loop(0, n_chunks, body, 0)
        plsc.subcore_barrier()
        pltpu.sync_copy(shared.at[pl.ds(s * zr, zr)],
                        out_hbm.at[c, pl.ds(s * zr, zr)])

    return k(vals, idx2d, zrows)


def _node_body(h_ref, p0_ref, p1_ref, w1h_ref, w1a_ref, b1_ref, w2_ref, b2_ref,
               g_ref, bln_ref, out_ref):
    h = h_ref[...]
    agg = p0_ref[...][:, :H] + p1_ref[...][:, :H]
    d1 = _silu(jnp.dot(h, w1h_ref[...], preferred_element_type=jnp.float32)
               + jnp.dot(agg, w1a_ref[...], preferred_element_type=jnp.float32)
               + b1_ref[...])
    dh = jnp.dot(d1, w2_ref[...], preferred_element_type=jnp.float32) + b2_ref[...]
    hn = h + dh
    mu = jnp.mean(hn, axis=-1, keepdims=True)
    var = jnp.mean((hn - mu) * (hn - mu), axis=-1, keepdims=True)
    out_ref[...] = (hn - mu) * lax.rsqrt(var + 1e-05) * g_ref[...] + bln_ref[...]


def _node_update(p, h, p0, p1, block=5000):
    N = h.shape[0]
    (w1, b1), (w2, b2) = p['phi_h']
    ew = lambda *_: (0, 0)
    out = pl.pallas_call(
        _node_body,
        grid=(N // block,),
        in_specs=[
            pl.BlockSpec((block, H), lambda i: (i, 0)),
            pl.BlockSpec((block, VW), lambda i: (i, 0)),
            pl.BlockSpec((block, VW), lambda i: (i, 0)),
            pl.BlockSpec((H, H), ew),
            pl.BlockSpec((H, H), ew),
            pl.BlockSpec((1, H), ew),
            pl.BlockSpec((H, H), ew),
            pl.BlockSpec((1, H), ew),
            pl.BlockSpec((1, H), ew),
            pl.BlockSpec((1, H), ew),
        ],
        out_specs=pl.BlockSpec((block, H), lambda i: (i, 0)),
        out_shape=jax.ShapeDtypeStruct((N, H), jnp.float32),
    )(h, p0, p1, w1[:H], w1[H:], b1[None, :], w2, b2[None, :],
      p['ln_g'][None, :], p['ln_b'][None, :])
    return out


def _egnn_layer(p, h, x, gidx2d, didx2d, zrows, nt, epad, edge_attr, N):
    comb = jnp.concatenate(
        [h, x, jnp.zeros((N, CW - H - 3), jnp.float32)], axis=1)
    gathered = _sc_gather(comb, gidx2d, epad)
    vals = _edge_mlp(p, gathered, edge_attr, epad)
    part = _sc_scatter(vals, didx2d, zrows, nt)
    h_out = _node_update(p, h, part[0, :N], part[1, :N])
    dx = part[0, :N, H:H + 3] + part[1, :N, H:H + 3]
    return h_out, x + dx


def _pool(h, batch, nb):
    s = jax.ops.segment_sum(h, batch, num_segments=nb)
    c = jax.ops.segment_sum(jnp.ones((h.shape[0], 1), dtype=h.dtype), batch, num_segments=nb)
    return s / jnp.clip(c, 1.0, None)


def _mlp(params, v):
    n = len(params)
    for i, (W, b) in enumerate(params):
        v = v @ W + b
        if i < n - 1:
            v = _silu(v)
    return v


def _graph_stack(layers, h, x, src, dst, edge_attr, N):
    E = src.shape[0]
    epad = ((E + NWORK * 8 * 128 - 1) // (NWORK * 8 * 128)) * (NWORK * 8 * 128)
    nt = ((N + 1 + 127) // 128) * 128
    dst_pad = jnp.pad(dst.astype(jnp.int32), (0, epad - E), constant_values=N)
    didx2d = dst_pad.reshape(epad // 128, 128)
    src_pad = jnp.pad(src.astype(jnp.int32), (0, epad - E), constant_values=0)
    gidx2d = jnp.concatenate([src_pad, jnp.minimum(dst_pad, N - 1)]
                             ).reshape(2 * epad // 128, 128)
    zrows = jnp.zeros((nt // 16, VW), jnp.float32)
    for p in layers:
        h, x = _egnn_layer(p, h, x, gidx2d, didx2d, zrows, nt, epad, edge_attr, N)
    return h, x


def kernel(prot_pos, prot_feat, prot_edge_index, prot_edge_attr, lig_pos, lig_feat,
           lig_edge_index, lig_edge_attr, prot_batch, lig_batch, priors,
           prot_proj_W, prot_proj_b, lig_proj_W, lig_proj_b,
           prot_layers, lig_layers, readout):
    NP = prot_pos.shape[0]
    NL = lig_pos.shape[0]
    h_p = prot_feat @ prot_proj_W + prot_proj_b
    h_p, _ = _graph_stack(prot_layers, h_p, prot_pos,
                          prot_edge_index[0], prot_edge_index[1],
                          prot_edge_attr, NP)
    h_l = lig_feat @ lig_proj_W + lig_proj_b
    h_l, _ = _graph_stack(lig_layers, h_l, lig_pos,
                          lig_edge_index[0], lig_edge_index[1],
                          lig_edge_attr, NL)
    B = priors.shape[0]
    prot_pool = _pool(h_p, prot_batch, B)
    lig_pool = _pool(h_l, lig_batch, B)
    complex_feat = jnp.concatenate([prot_pool, lig_pool, priors], axis=-1)
    pred = _mlp(readout, complex_feat)
    return pred[:, 0]


# speedup vs baseline: 3.9201x; 1.0890x over previous
import functools

import jax
import jax.numpy as jnp
from jax import lax
from jax.experimental import pallas as pl
from jax.experimental.pallas import tpu as pltpu
from jax.experimental.pallas import tpu_sc as plsc

H = 32
ED = 16
VW = 40
CH = 1024
NWORK = 32


def _silu(v):
    return v * jax.nn.sigmoid(v)


CW = 48


def _sc_gather(comb, idxflat, epad):
    total = idxflat.shape[0]
    per_w = total // NWORK
    n_chunks = per_w // 1024
    mesh = plsc.VectorSubcoreMesh(core_axis_name="c", subcore_axis_name="s")

    @functools.partial(
        pl.kernel,
        mesh=mesh,
        compiler_params=pltpu.CompilerParams(use_tc_tiling_on_sc=False),
        out_type=jax.ShapeDtypeStruct((2 * epad, CW), jnp.float32),
        scratch_types=[
            pltpu.VMEM((1024,), jnp.int32),
            pltpu.VMEM((1024, CW), jnp.float32),
        ],
    )
    def k(comb_hbm, idx_hbm, out_hbm, idx_v, rows_v):
        c = lax.axis_index("c")
        s = lax.axis_index("s")
        wid = s * 2 + c

        def body(t, carry):
            e0 = wid * per_w + t * 1024
            pltpu.sync_copy(idx_hbm.at[pl.ds(e0, 1024)], idx_v)
            pltpu.sync_copy(comb_hbm.at[idx_v], rows_v)
            pltpu.sync_copy(rows_v, out_hbm.at[pl.ds(e0, 1024)])
            return carry

        lax.fori_loop(0, n_chunks, body, 0)

    out = k(comb, idxflat)
    return out.reshape(2, epad, CW)


def _edge_body(g0_ref, g1_ref, ea_ref,
               w1h_i_ref, w1h_j_ref, w1e_ref, w1d_ref, b1_ref,
               w2_ref, b2_ref, wx1_ref, bx1_ref, wx2_ref, bx2_ref,
               out_ref):
    g0 = g0_ref[0]
    g1 = g1_ref[0]
    hi = g0[:, :H]
    hj = g1[:, :H]
    xi = g0[:, H:H + 3]
    xj = g1[:, H:H + 3]
    diff = xj - xi
    sq = jnp.sum(diff * diff, axis=-1, keepdims=True)
    dist = jnp.sqrt(sq + 1e-09)
    dn3 = diff / (jnp.sqrt(sq) + 1e-09)
    ea = ea_ref[...]
    pre1 = (jnp.dot(hi, w1h_i_ref[...], preferred_element_type=jnp.float32)
            + jnp.dot(hj, w1h_j_ref[...], preferred_element_type=jnp.float32)
            + jnp.dot(ea, w1e_ref[...], preferred_element_type=jnp.float32)
            + dist * w1d_ref[...]
            + b1_ref[...])
    m1 = _silu(pre1)
    m = jnp.dot(m1, w2_ref[...], preferred_element_type=jnp.float32) + b2_ref[...]
    gg = _silu(jnp.dot(m, wx1_ref[...], preferred_element_type=jnp.float32) + bx1_ref[...])
    gate = jnp.dot(gg, wx2_ref[...], preferred_element_type=jnp.float32) + bx2_ref[...]
    cmsg = dn3 * gate
    blk = hi.shape[0]
    out_ref[...] = jnp.concatenate(
        [m, cmsg, jnp.zeros((blk, VW - H - 3), jnp.float32)], axis=1)


def _edge_mlp(p, gathered, ea, epad, block=8000):
    E = ea.shape[0]
    (w1, b1), (w2, b2) = p['phi_e']
    (wx1, bx1), (wx2, bx2) = p['phi_x']
    ew = lambda *_: (0, 0)
    out = pl.pallas_call(
        _edge_body,
        grid=(E // block,),
        in_specs=[
            pl.BlockSpec((1, block, CW), lambda i: (0, i, 0)),
            pl.BlockSpec((1, block, CW), lambda i: (1, i, 0)),
            pl.BlockSpec((block, ED), lambda i: (i, 0)),
            pl.BlockSpec((H, H), ew),
            pl.BlockSpec((H, H), ew),
            pl.BlockSpec((ED, H), ew),
            pl.BlockSpec((1, H), ew),
            pl.BlockSpec((1, H), ew),
            pl.BlockSpec((H, H), ew),
            pl.BlockSpec((1, H), ew),
            pl.BlockSpec((H, H), ew),
            pl.BlockSpec((1, H), ew),
            pl.BlockSpec((H, 1), ew),
            pl.BlockSpec((1, 1), ew),
        ],
        out_specs=pl.BlockSpec((block, VW), lambda i: (i, 0)),
        out_shape=jax.ShapeDtypeStruct((epad, VW), jnp.float32),
    )(gathered, gathered, ea,
      w1[:H], w1[H:2 * H], w1[2 * H:2 * H + ED], w1[2 * H + ED:], b1[None, :],
      w2, b2[None, :], wx1, bx1[None, :], wx2, bx2[None, :])
    return out


def _sc_scatter(vals, idx2d, zrows, nt):
    rows = idx2d.shape[0]
    per_w = rows // NWORK
    n_chunks = per_w // 2
    zr = nt // 16
    mesh = plsc.VectorSubcoreMesh(core_axis_name="c", subcore_axis_name="s")

    @functools.partial(
        pl.kernel,
        mesh=mesh,
        compiler_params=pltpu.CompilerParams(use_tc_tiling_on_sc=False),
        out_type=jax.ShapeDtypeStruct((2, nt, VW), jnp.float32),
        scratch_types=[
            pltpu.VMEM((2, 128), jnp.int32),
            pltpu.VMEM((128, VW), jnp.float32),
            pltpu.VMEM_SHARED((nt, VW), jnp.float32),
        ],
    )
    def k(vals_hbm, idx_hbm, z_hbm, out_hbm, idx_v, vals_v, shared):
        c = lax.axis_index("c")
        s = lax.axis_index("s")
        wid = s * 2 + c
        pltpu.sync_copy(z_hbm, shared.at[pl.ds(s * zr, zr)])
        plsc.subcore_barrier()

        def body(t, carry):
            row0 = wid * per_w + t * 2
            pltpu.sync_copy(idx_hbm.at[pl.ds(row0, 2)], idx_v)
            for j in range(2):
                pltpu.sync_copy(vals_hbm.at[pl.ds((row0 + j) * 128, 128)], vals_v)
                pltpu.sync_copy(vals_v, shared.at[idx_v.at[j]], add=True)
            return carry

        lax.fori_loop(0, n_chunks, body, 0)
        plsc.subcore_barrier()
        pltpu.sync_copy(shared.at[pl.ds(s * zr, zr)],
                        out_hbm.at[c, pl.ds(s * zr, zr)])

    return k(vals, idx2d, zrows)


def _node_body(h_ref, p0_ref, p1_ref, w1h_ref, w1a_ref, b1_ref, w2_ref, b2_ref,
               g_ref, bln_ref, out_ref):
    h = h_ref[...]
    agg = p0_ref[...][:, :H] + p1_ref[...][:, :H]
    d1 = _silu(jnp.dot(h, w1h_ref[...], preferred_element_type=jnp.float32)
               + jnp.dot(agg, w1a_ref[...], preferred_element_type=jnp.float32)
               + b1_ref[...])
    dh = jnp.dot(d1, w2_ref[...], preferred_element_type=jnp.float32) + b2_ref[...]
    hn = h + dh
    mu = jnp.mean(hn, axis=-1, keepdims=True)
    var = jnp.mean((hn - mu) * (hn - mu), axis=-1, keepdims=True)
    out_ref[...] = (hn - mu) * lax.rsqrt(var + 1e-05) * g_ref[...] + bln_ref[...]


def _node_update(p, h, p0, p1, block=5000):
    N = h.shape[0]
    (w1, b1), (w2, b2) = p['phi_h']
    ew = lambda *_: (0, 0)
    out = pl.pallas_call(
        _node_body,
        grid=(N // block,),
        in_specs=[
            pl.BlockSpec((block, H), lambda i: (i, 0)),
            pl.BlockSpec((block, VW), lambda i: (i, 0)),
            pl.BlockSpec((block, VW), lambda i: (i, 0)),
            pl.BlockSpec((H, H), ew),
            pl.BlockSpec((H, H), ew),
            pl.BlockSpec((1, H), ew),
            pl.BlockSpec((H, H), ew),
            pl.BlockSpec((1, H), ew),
            pl.BlockSpec((1, H), ew),
            pl.BlockSpec((1, H), ew),
        ],
        out_specs=pl.BlockSpec((block, H), lambda i: (i, 0)),
        out_shape=jax.ShapeDtypeStruct((N, H), jnp.float32),
    )(h, p0, p1, w1[:H], w1[H:], b1[None, :], w2, b2[None, :],
      p['ln_g'][None, :], p['ln_b'][None, :])
    return out


def _egnn_layer(p, h, x, gidx, didx2d, zrows, nt, epad, edge_attr, N):
    comb = jnp.concatenate(
        [h, x, jnp.zeros((N, CW - H - 3), jnp.float32)], axis=1)
    gathered = _sc_gather(comb, gidx, epad)
    vals = _edge_mlp(p, gathered, edge_attr, epad)
    part = _sc_scatter(vals, didx2d, zrows, nt)
    h_out = _node_update(p, h, part[0, :N], part[1, :N])
    dx = part[0, :N, H:H + 3] + part[1, :N, H:H + 3]
    return h_out, x + dx


def _pool(h, batch, nb):
    s = jax.ops.segment_sum(h, batch, num_segments=nb)
    c = jax.ops.segment_sum(jnp.ones((h.shape[0], 1), dtype=h.dtype), batch, num_segments=nb)
    return s / jnp.clip(c, 1.0, None)


def _mlp(params, v):
    n = len(params)
    for i, (W, b) in enumerate(params):
        v = v @ W + b
        if i < n - 1:
            v = _silu(v)
    return v


def _graph_stack(layers, h, x, src, dst, edge_attr, N):
    E = src.shape[0]
    epad = ((E + NWORK * 8 * 128 - 1) // (NWORK * 8 * 128)) * (NWORK * 8 * 128)
    nt = ((N + 1 + 127) // 128) * 128
    dst_pad = jnp.pad(dst.astype(jnp.int32), (0, epad - E), constant_values=N)
    didx2d = dst_pad.reshape(epad // 128, 128)
    src_pad = jnp.pad(src.astype(jnp.int32), (0, epad - E), constant_values=0)
    gidx = jnp.concatenate([src_pad, jnp.minimum(dst_pad, N - 1)])
    zrows = jnp.zeros((nt // 16, VW), jnp.float32)
    for p in layers:
        h, x = _egnn_layer(p, h, x, gidx, didx2d, zrows, nt, epad, edge_attr, N)
    return h, x


def kernel(prot_pos, prot_feat, prot_edge_index, prot_edge_attr, lig_pos, lig_feat,
           lig_edge_index, lig_edge_attr, prot_batch, lig_batch, priors,
           prot_proj_W, prot_proj_b, lig_proj_W, lig_proj_b,
           prot_layers, lig_layers, readout):
    NP = prot_pos.shape[0]
    NL = lig_pos.shape[0]
    h_p = prot_feat @ prot_proj_W + prot_proj_b
    h_p, _ = _graph_stack(prot_layers, h_p, prot_pos,
                          prot_edge_index[0], prot_edge_index[1],
                          prot_edge_attr, NP)
    h_l = lig_feat @ lig_proj_W + lig_proj_b
    h_l, _ = _graph_stack(lig_layers, h_l, lig_pos,
                          lig_edge_index[0], lig_edge_index[1],
                          lig_edge_attr, NL)
    B = priors.shape[0]
    prot_pool = _pool(h_p, prot_batch, B)
    lig_pool = _pool(h_l, lig_batch, B)
    complex_feat = jnp.concatenate([prot_pool, lig_pool, priors], axis=-1)
    pred = _mlp(readout, complex_feat)
    return pred[:, 0]
